# Initial kernel scaffold; baseline (speedup 1.0000x reference)
import jax
import jax.numpy as jnp
from jax.experimental import pallas as pl


def kernel(vis_emb, query_emb, W):
    raise NotImplementedError("write your pallas kernel here")


# speedup vs baseline: 1.8116x; 1.8116x over previous
import functools

import jax
import jax.numpy as jnp
from jax import lax
from jax.experimental import pallas as pl

_TOP_K = 8


def _router_body(repeat, q_ref, w1_ref, w2_ref, x_ref,
                 logits_ref, scores_ref, tks_ref, tki_ref):
    i = pl.program_id(0)
    x = x_ref[0].astype(jnp.bfloat16)
    l = jnp.dot(x, w1_ref[...].astype(jnp.bfloat16),
                preferred_element_type=jnp.float32)
    qb4 = jnp.dot(q_ref[...].astype(jnp.bfloat16),
                  w2_ref[...].astype(jnp.bfloat16),
                  preferred_element_type=jnp.float32)
    g = i // repeat
    row_iota = lax.broadcasted_iota(jnp.int32, qb4.shape, 0)
    qb = jnp.sum(jnp.where(row_iota == g, qb4, 0.0), axis=0, keepdims=True)
    l = l + qb
    logits_ref[0] = l

    m = jnp.max(l, axis=-1, keepdims=True)
    ex = jnp.exp(l - m)
    sm = jnp.sum(ex, axis=-1, keepdims=True)
    sc = ex / sm
    scores_ref[0] = sc

    e_iota = lax.broadcasted_iota(jnp.int32, sc.shape, 1)
    key = (lax.bitcast_convert_type(sc, jnp.int32) & ~0x3F) | (0x3F - e_iota)
    ks, ki = [], []
    for _ in range(_TOP_K):
        mx = jnp.max(key, axis=-1, keepdims=True)
        key = jnp.where(key == mx, 0, key)
        ki.append(0x3F - (mx & 0x3F))
        ks.append(lax.bitcast_convert_type(mx & ~0x3F, jnp.float32))
    tks_ref[0] = jnp.concatenate(ks, axis=-1)
    tki_ref[0] = jnp.concatenate(ki, axis=-1)


def kernel(vis_emb, query_emb, W):
    total, T, H = vis_emb.shape
    B = query_emb.shape[0]
    E = W.shape[0]
    repeat = total // B
    w1t = W[:, :H].T
    w2t = W[:, H:].T

    out_shape = (
        jax.ShapeDtypeStruct((total, T, E), jnp.float32),
        jax.ShapeDtypeStruct((total, T, E), jnp.float32),
        jax.ShapeDtypeStruct((total, T, _TOP_K), jnp.float32),
        jax.ShapeDtypeStruct((total, T, _TOP_K), jnp.int32),
    )
    logits, scores, tks, tki = pl.pallas_call(
        functools.partial(_router_body, repeat),
        grid=(total,),
        in_specs=[
            pl.BlockSpec((B, H), lambda i: (0, 0)),
            pl.BlockSpec((H, E), lambda i: (0, 0)),
            pl.BlockSpec((H, E), lambda i: (0, 0)),
            pl.BlockSpec((1, T, H), lambda i: (i, 0, 0)),
        ],
        out_specs=[
            pl.BlockSpec((1, T, E), lambda i: (i, 0, 0)),
            pl.BlockSpec((1, T, E), lambda i: (i, 0, 0)),
            pl.BlockSpec((1, T, _TOP_K), lambda i: (i, 0, 0)),
            pl.BlockSpec((1, T, _TOP_K), lambda i: (i, 0, 0)),
        ],
        out_shape=out_shape,
    )(query_emb, w1t, w2t, vis_emb)
    return (tks, tki, scores, logits)
